# baseline (speedup 1.0000x reference)
import functools
import jax
import jax.numpy as jnp
from jax.experimental import pallas as pl
from jax.experimental.pallas import tpu as pltpu

H = 128
BE = 2048


def _edge_mlp_body(za_ref, zb_ref, w_ref, b_ref, kh_ref, f23_ref, kc_ref):
    h = jax.nn.relu(za_ref[...] + zb_ref[...])
    u = jnp.dot(h, w_ref[...], preferred_element_type=jnp.float32) + b_ref[...]
    k = jax.nn.sigmoid(u[:, 256:257])
    kh_ref[...] = h * k
    f23_ref[...] = u[:, 0:256] * k
    ones = jnp.ones_like(k)
    zeros = jnp.zeros((k.shape[0], 14), jnp.float32)
    kc_ref[...] = jnp.concatenate([k, ones, zeros], axis=1)


def _edge_mlp(za, zb, w23g, b23g):
    e = za.shape[0]
    grid = (e // BE,)
    return pl.pallas_call(
        _edge_mlp_body,
        grid=grid,
        in_specs=[
            pl.BlockSpec((BE, 256), lambda i: (i, 0)),
            pl.BlockSpec((BE, 256), lambda i: (i, 0)),
            pl.BlockSpec((256, 384), lambda i: (0, 0)),
            pl.BlockSpec((1, 384), lambda i: (0, 0)),
        ],
        out_specs=[
            pl.BlockSpec((BE, 256), lambda i: (i, 0)),
            pl.BlockSpec((BE, 256), lambda i: (i, 0)),
            pl.BlockSpec((BE, 16), lambda i: (i, 0)),
        ],
        out_shape=[
            jax.ShapeDtypeStruct((e, 256), jnp.float32),
            jax.ShapeDtypeStruct((e, 256), jnp.float32),
            jax.ShapeDtypeStruct((e, 16), jnp.float32),
        ],
    )(za, zb, w23g, b23g)


def _pack_w23g(W2, b2):
    w = jnp.concatenate([W2[:, 129:385], W2[:, 0:1],
                         jnp.zeros((256, 127), jnp.float32)], axis=1)
    b = jnp.concatenate([b2[129:385], b2[0:1],
                         jnp.zeros((127,), jnp.float32)])[None, :]
    return w, b


def _segment_side(kh, f23, kc, dst, n):
    khs = jax.ops.segment_sum(kh, dst, num_segments=n)
    kcs = jax.ops.segment_sum(kc, dst, num_segments=n)
    mx = jax.ops.segment_max(f23[:, 0:128], dst, num_segments=n)
    mn = jax.ops.segment_min(f23[:, 128:256], dst, num_segments=n)
    return khs, kcs, mx, mn


def _node_combine(x_gc, khs, kcs, mx, mn, W2, b2, Wr, br):
    ksum = kcs[:, 0:1]
    cnt = kcs[:, 1:2]
    has = cnt > 0
    n1 = khs @ W2[:, 1:129] + ksum * b2[1:129]
    n4 = (khs @ W2[:, 385:513] + ksum * b2[385:513]) / jnp.maximum(cnt, 1.0)
    n2 = jnp.where(has, mx, 0.0)
    n3 = jnp.where(has, mn, 0.0)
    return jnp.concatenate([x_gc, n1, n2, n3, n4], axis=1) @ Wr + br


def kernel(nf_gc, nf_gs0, nf_gs1, edge_index_cc, edge_index_s2c, Wcc1, bcc1,
           Wcc2, bcc2, Wrcc, brcc, Ws1, bs1, Ws2, bs2, Wrs, brs, Wgc, bgc,
           Wgs, bgs):
    x_gc = nf_gc
    x_gs = jnp.concatenate([nf_gs0, nf_gs1], axis=1)
    n = x_gc.shape[0]

    A_cc = x_gc @ Wcc1[0:128] + bcc1
    B_cc = x_gc @ Wcc1[128:256]
    A_s = x_gs @ Ws1[0:256] + bs1
    B_s = x_gc @ Ws1[256:384]

    w23g_cc, b23g_cc = _pack_w23g(Wcc2, bcc2)
    w23g_s, b23g_s = _pack_w23g(Ws2, bs2)

    src_cc, dst_cc = edge_index_cc[0], edge_index_cc[1]
    src_s, dst_s = edge_index_s2c[0], edge_index_s2c[1]

    kh_cc, f23_cc, kc_cc = _edge_mlp(A_cc[src_cc], B_cc[dst_cc],
                                     w23g_cc, b23g_cc)
    kh_s, f23_s, kc_s = _edge_mlp(A_s[src_s], B_s[dst_s], w23g_s, b23g_s)

    khs_cc, kcs_cc, mx_cc, mn_cc = _segment_side(kh_cc, f23_cc, kc_cc,
                                                 dst_cc, n)
    khs_s, kcs_s, mx_s, mn_s = _segment_side(kh_s, f23_s, kc_s, dst_s, n)

    new_ccx = _node_combine(x_gc, khs_cc, kcs_cc, mx_cc, mn_cc,
                            Wcc2, bcc2, Wrcc, brcc)
    new_cx = _node_combine(x_gc, khs_s, kcs_s, mx_s, mn_s,
                           Ws2, bs2, Wrs, brs)

    out_fc = (new_ccx + new_cx) @ Wgc + bgc
    out_fs = x_gs @ Wgs + bgs
    return out_fc, out_fs
